# baseline (device time: 16054 ns/iter reference)
import jax
import jax.numpy as jnp
from jax import lax
from jax.experimental import pallas as pl
from jax.experimental.pallas import tpu as pltpu

N_DEV = 8


def kernel(Q, K, V):
    b, sq, h, d = Q.shape
    skv = K.shape[1]
    scale = d ** -0.5
    nb = b // 4

    Kt = jnp.transpose(K, (0, 2, 3, 1))
    Vt = jnp.transpose(V, (0, 2, 3, 1))
    Qs = Q[:, 0, :, :]

    def body(q_ref, k_ref, v_ref, out_ref,
             kv_vmem, pk_ref, recv_ref, copy_sems, send_sems, recv_sem):
        my_x = lax.axis_index("x")
        my_y = lax.axis_index("y")
        my_z = lax.axis_index("z")
        my_lin = my_x * 4 + my_y * 2 + my_z
        gid = my_x * 2 + my_y
        b0 = gid * nb

        kdma = pltpu.make_async_copy(
            k_ref.at[pl.ds(b0, nb)], kv_vmem.at[0], copy_sems.at[0])
        vdma = pltpu.make_async_copy(
            v_ref.at[pl.ds(b0, nb)], kv_vmem.at[1], copy_sems.at[1])
        kdma.start()
        vdma.start()

        barrier_sem = pltpu.get_barrier_semaphore()
        for peer in range(N_DEV):
            @pl.when(peer != my_lin)
            def _():
                pl.semaphore_signal(
                    barrier_sem, inc=1,
                    device_id=peer, device_id_type=pl.DeviceIdType.LOGICAL,
                )
        pl.semaphore_wait(barrier_sem, N_DEV - 1)

        kdma.wait()
        vdma.wait()

        q = q_ref[pl.ds(b0, nb)]
        k = kv_vmem[0]
        v = kv_vmem[1]
        s = jnp.sum(q[:, :, :, None] * k, axis=2) * scale
        m = jnp.max(s, axis=2, keepdims=True)
        p = jnp.exp(s - m)
        l = jnp.sum(p, axis=2, keepdims=True)
        o = jnp.sum(p[:, :, None, :] * v, axis=3)

        pk_ref[:, :, 0:d] = o
        pk_ref[:, :, d:d + 1] = m
        pk_ref[:, :, d + 1:d + 2] = l

        for peer in range(N_DEV):
            @pl.when(peer != my_lin)
            def _():
                rdma = pltpu.make_async_remote_copy(
                    src_ref=pk_ref,
                    dst_ref=recv_ref.at[my_lin],
                    send_sem=send_sems.at[peer],
                    recv_sem=recv_sem,
                    device_id=peer,
                    device_id_type=pl.DeviceIdType.LOGICAL,
                )
                rdma.start()

        for _ in range(N_DEV - 1):
            recv_wait = pltpu.make_async_remote_copy(
                src_ref=pk_ref,
                dst_ref=recv_ref.at[0],
                send_sem=send_sems.at[0],
                recv_sem=recv_sem,
                device_id=0,
                device_id_type=pl.DeviceIdType.LOGICAL,
            )
            recv_wait.wait_recv()

        own = pk_ref[...]
        for g in range(4):
            def slot(s_idx):
                data = recv_ref[s_idx]
                data = jnp.where(s_idx == my_lin, own, data)
                return (data[:, :, 0:d], data[:, :, d:d + 1],
                        data[:, :, d + 1:d + 2])
            o1, m1, l1 = slot(2 * g)
            o2, m2, l2 = slot(2 * g + 1)
            mn = jnp.maximum(m1, m2)
            a1 = jnp.exp(m1 - mn)
            a2 = jnp.exp(m2 - mn)
            ln = a1 * l1 + a2 * l2
            on = (a1 * o1 + a2 * o2) / ln
            out_ref[pl.ds(g * nb, nb)] = on[:, None, :, :]

        for peer in range(N_DEV):
            @pl.when(peer != my_lin)
            def _():
                drain = pltpu.make_async_remote_copy(
                    src_ref=pk_ref,
                    dst_ref=recv_ref.at[my_lin],
                    send_sem=send_sems.at[peer],
                    recv_sem=recv_sem,
                    device_id=peer,
                    device_id_type=pl.DeviceIdType.LOGICAL,
                )
                drain.wait_send()

    return pl.pallas_call(
        body,
        out_shape=jax.ShapeDtypeStruct((b, sq, h, d), jnp.float32),
        in_specs=[
            pl.BlockSpec(memory_space=pltpu.VMEM),
            pl.BlockSpec(memory_space=pltpu.MemorySpace.HBM),
            pl.BlockSpec(memory_space=pltpu.MemorySpace.HBM),
        ],
        out_specs=pl.BlockSpec(memory_space=pltpu.VMEM),
        scratch_shapes=[
            pltpu.VMEM((2, nb, h, d, skv), jnp.float32),
            pltpu.VMEM((nb, h, d + 2), jnp.float32),
            pltpu.VMEM((N_DEV, nb, h, d + 2), jnp.float32),
            pltpu.SemaphoreType.DMA((2,)),
            pltpu.SemaphoreType.DMA((N_DEV,)),
            pltpu.SemaphoreType.DMA,
        ],
        compiler_params=pltpu.CompilerParams(collective_id=0),
    )(Qs, Kt, Vt)


# device time: 10351 ns/iter; 1.5510x vs baseline; 1.5510x over previous
import jax
import jax.numpy as jnp
from jax import lax
from jax.experimental import pallas as pl
from jax.experimental.pallas import tpu as pltpu

N_DEV = 8


def kernel(Q, K, V):
    b, sq, h, d = Q.shape
    skv = K.shape[1]
    scale = d ** -0.5
    nb = b // 4

    Kt = jnp.transpose(K, (0, 2, 3, 1))
    Vt = jnp.transpose(V, (0, 2, 3, 1))
    Qs = Q[:, 0, :, :]
    Kt = pltpu.with_memory_space_constraint(Kt, pltpu.MemorySpace.HBM)
    Vt = pltpu.with_memory_space_constraint(Vt, pltpu.MemorySpace.HBM)

    def body(q_ref, k_ref, v_ref, out_ref,
             kv_vmem, pk_ref, recv_ref, copy_sems, send_sems, recv_sem):
        my_x = lax.axis_index("x")
        my_y = lax.axis_index("y")
        my_z = lax.axis_index("z")
        my_lin = my_x * 4 + my_y * 2 + my_z
        gid = my_x * 2 + my_y
        b0 = gid * nb

        kdma = pltpu.make_async_copy(
            k_ref.at[pl.ds(b0, nb)], kv_vmem.at[0], copy_sems.at[0])
        vdma = pltpu.make_async_copy(
            v_ref.at[pl.ds(b0, nb)], kv_vmem.at[1], copy_sems.at[1])
        kdma.start()
        vdma.start()

        barrier_sem = pltpu.get_barrier_semaphore()
        for peer in range(N_DEV):
            @pl.when(peer != my_lin)
            def _():
                pl.semaphore_signal(
                    barrier_sem, inc=1,
                    device_id=peer, device_id_type=pl.DeviceIdType.LOGICAL,
                )
        pl.semaphore_wait(barrier_sem, N_DEV - 1)

        kdma.wait()
        vdma.wait()

        q = q_ref[pl.ds(b0, nb)]
        k = kv_vmem[0]
        v = kv_vmem[1]
        s = jnp.sum(q[:, :, :, None] * k, axis=2) * scale
        m = jnp.max(s, axis=2, keepdims=True)
        p = jnp.exp(s - m)
        l = jnp.sum(p, axis=2, keepdims=True)
        o = jnp.sum(p[:, :, None, :] * v, axis=3)

        pk_ref[:, :, 0:d] = o
        pk_ref[:, :, d:d + 1] = m
        pk_ref[:, :, d + 1:d + 2] = l

        for peer in range(N_DEV):
            @pl.when(peer != my_lin)
            def _():
                rdma = pltpu.make_async_remote_copy(
                    src_ref=pk_ref,
                    dst_ref=recv_ref.at[my_lin],
                    send_sem=send_sems.at[peer],
                    recv_sem=recv_sem,
                    device_id=peer,
                    device_id_type=pl.DeviceIdType.LOGICAL,
                )
                rdma.start()

        for _ in range(N_DEV - 1):
            recv_wait = pltpu.make_async_remote_copy(
                src_ref=pk_ref,
                dst_ref=recv_ref.at[0],
                send_sem=send_sems.at[0],
                recv_sem=recv_sem,
                device_id=0,
                device_id_type=pl.DeviceIdType.LOGICAL,
            )
            recv_wait.wait_recv()

        own = pk_ref[...]
        for g in range(4):
            def slot(s_idx):
                data = recv_ref[s_idx]
                data = jnp.where(s_idx == my_lin, own, data)
                return (data[:, :, 0:d], data[:, :, d:d + 1],
                        data[:, :, d + 1:d + 2])
            o1, m1, l1 = slot(2 * g)
            o2, m2, l2 = slot(2 * g + 1)
            mn = jnp.maximum(m1, m2)
            a1 = jnp.exp(m1 - mn)
            a2 = jnp.exp(m2 - mn)
            ln = a1 * l1 + a2 * l2
            on = (a1 * o1 + a2 * o2) / ln
            out_ref[pl.ds(g * nb, nb)] = on[:, None, :, :]

        for peer in range(N_DEV):
            @pl.when(peer != my_lin)
            def _():
                drain = pltpu.make_async_remote_copy(
                    src_ref=pk_ref,
                    dst_ref=recv_ref.at[my_lin],
                    send_sem=send_sems.at[peer],
                    recv_sem=recv_sem,
                    device_id=peer,
                    device_id_type=pl.DeviceIdType.LOGICAL,
                )
                drain.wait_send()

    return pl.pallas_call(
        body,
        out_shape=jax.ShapeDtypeStruct((b, sq, h, d), jnp.float32),
        in_specs=[
            pl.BlockSpec(memory_space=pltpu.VMEM),
            pl.BlockSpec(memory_space=pltpu.MemorySpace.HBM),
            pl.BlockSpec(memory_space=pltpu.MemorySpace.HBM),
        ],
        out_specs=pl.BlockSpec(memory_space=pltpu.VMEM),
        scratch_shapes=[
            pltpu.VMEM((2, nb, h, d, skv), jnp.float32),
            pltpu.VMEM((nb, h, d + 2), jnp.float32),
            pltpu.VMEM((N_DEV, nb, h, d + 2), jnp.float32),
            pltpu.SemaphoreType.DMA((2,)),
            pltpu.SemaphoreType.DMA((N_DEV,)),
            pltpu.SemaphoreType.DMA,
        ],
        compiler_params=pltpu.CompilerParams(collective_id=0),
    )(Qs, Kt, Vt)


# device time: 9624 ns/iter; 1.6681x vs baseline; 1.0755x over previous
import jax
import jax.numpy as jnp
from jax import lax
from jax.experimental import pallas as pl
from jax.experimental.pallas import tpu as pltpu

N_DEV = 8


def kernel(Q, K, V):
    b, sq, h, d = Q.shape
    skv = K.shape[1]
    scale = d ** -0.5
    nb = b // 4

    Kt = jnp.transpose(K, (0, 2, 3, 1))
    Vt = jnp.transpose(V, (0, 2, 3, 1))
    Qs = Q[:, 0, :, :]
    Kt = pltpu.with_memory_space_constraint(Kt, pltpu.MemorySpace.HBM)
    Vt = pltpu.with_memory_space_constraint(Vt, pltpu.MemorySpace.HBM)
    Qs = pltpu.with_memory_space_constraint(Qs, pltpu.MemorySpace.HBM)

    def body(q_ref, k_ref, v_ref, out_ref,
             kv_vmem, q_vmem, pk_ref, recv_ref, copy_sems, send_sems,
             recv_sem):
        my_x = lax.axis_index("x")
        my_y = lax.axis_index("y")
        my_z = lax.axis_index("z")
        my_lin = my_x * 4 + my_y * 2 + my_z
        gid = my_x * 2 + my_y
        b0 = gid * nb

        kdma = pltpu.make_async_copy(
            k_ref.at[pl.ds(b0, nb)], kv_vmem.at[0], copy_sems.at[0])
        vdma = pltpu.make_async_copy(
            v_ref.at[pl.ds(b0, nb)], kv_vmem.at[1], copy_sems.at[1])
        qdma = pltpu.make_async_copy(
            q_ref.at[pl.ds(b0, nb)], q_vmem, copy_sems.at[2])
        kdma.start()
        vdma.start()
        qdma.start()

        barrier_sem = pltpu.get_barrier_semaphore()
        for peer in range(N_DEV):
            @pl.when(peer != my_lin)
            def _():
                pl.semaphore_signal(
                    barrier_sem, inc=1,
                    device_id=peer, device_id_type=pl.DeviceIdType.LOGICAL,
                )

        kdma.wait()
        vdma.wait()
        qdma.wait()

        q = q_vmem[...]
        k = kv_vmem[0]
        v = kv_vmem[1]
        s = jnp.sum(q[:, :, :, None] * k, axis=2) * scale
        m = jnp.max(s, axis=2, keepdims=True)
        p = jnp.exp(s - m)
        l = jnp.sum(p, axis=2, keepdims=True)
        o = jnp.sum(p[:, :, None, :] * v, axis=3)

        pk_ref[:, :, 0:d] = o
        pk_ref[:, :, d:d + 1] = m
        pk_ref[:, :, d + 1:d + 2] = l

        pl.semaphore_wait(barrier_sem, N_DEV - 1)

        for peer in range(N_DEV):
            @pl.when(peer != my_lin)
            def _():
                rdma = pltpu.make_async_remote_copy(
                    src_ref=pk_ref,
                    dst_ref=recv_ref.at[my_lin],
                    send_sem=send_sems.at[peer],
                    recv_sem=recv_sem,
                    device_id=peer,
                    device_id_type=pl.DeviceIdType.LOGICAL,
                )
                rdma.start()

        for _ in range(N_DEV - 1):
            recv_wait = pltpu.make_async_remote_copy(
                src_ref=pk_ref,
                dst_ref=recv_ref.at[0],
                send_sem=send_sems.at[0],
                recv_sem=recv_sem,
                device_id=0,
                device_id_type=pl.DeviceIdType.LOGICAL,
            )
            recv_wait.wait_recv()

        own = pk_ref[...]
        for g in range(4):
            def slot(s_idx):
                data = recv_ref[s_idx]
                data = jnp.where(s_idx == my_lin, own, data)
                return (data[:, :, 0:d], data[:, :, d:d + 1],
                        data[:, :, d + 1:d + 2])
            o1, m1, l1 = slot(2 * g)
            o2, m2, l2 = slot(2 * g + 1)
            mn = jnp.maximum(m1, m2)
            a1 = jnp.exp(m1 - mn)
            a2 = jnp.exp(m2 - mn)
            ln = a1 * l1 + a2 * l2
            on = (a1 * o1 + a2 * o2) / ln
            out_ref[pl.ds(g * nb, nb)] = on[:, None, :, :]

        for peer in range(N_DEV):
            @pl.when(peer != my_lin)
            def _():
                drain = pltpu.make_async_remote_copy(
                    src_ref=pk_ref,
                    dst_ref=recv_ref.at[my_lin],
                    send_sem=send_sems.at[peer],
                    recv_sem=recv_sem,
                    device_id=peer,
                    device_id_type=pl.DeviceIdType.LOGICAL,
                )
                drain.wait_send()

    return pl.pallas_call(
        body,
        out_shape=jax.ShapeDtypeStruct((b, sq, h, d), jnp.float32),
        in_specs=[
            pl.BlockSpec(memory_space=pltpu.MemorySpace.HBM),
            pl.BlockSpec(memory_space=pltpu.MemorySpace.HBM),
            pl.BlockSpec(memory_space=pltpu.MemorySpace.HBM),
        ],
        out_specs=pl.BlockSpec(memory_space=pltpu.VMEM),
        scratch_shapes=[
            pltpu.VMEM((2, nb, h, d, skv), jnp.float32),
            pltpu.VMEM((nb, h, d), jnp.float32),
            pltpu.VMEM((nb, h, d + 2), jnp.float32),
            pltpu.VMEM((N_DEV, nb, h, d + 2), jnp.float32),
            pltpu.SemaphoreType.DMA((3,)),
            pltpu.SemaphoreType.DMA((N_DEV,)),
            pltpu.SemaphoreType.DMA,
        ],
        compiler_params=pltpu.CompilerParams(collective_id=0),
    )(Qs, Kt, Vt)


# device time: 9620 ns/iter; 1.6688x vs baseline; 1.0004x over previous
import jax
import jax.numpy as jnp
from jax import lax
from jax.experimental import pallas as pl
from jax.experimental.pallas import tpu as pltpu

N_DEV = 8


def kernel(Q, K, V):
    b, sq, h, d = Q.shape
    skv = K.shape[1]
    scale = d ** -0.5
    nb = b // 4

    Kt = jnp.transpose(K, (0, 2, 3, 1))
    Vt = jnp.transpose(V, (0, 2, 3, 1))
    Qs = Q[:, 0, :, :]
    Kt = pltpu.with_memory_space_constraint(Kt, pltpu.MemorySpace.HBM)
    Vt = pltpu.with_memory_space_constraint(Vt, pltpu.MemorySpace.HBM)
    Qs = pltpu.with_memory_space_constraint(Qs, pltpu.MemorySpace.HBM)

    def body(q_ref, k_ref, v_ref, out_ref,
             kv_vmem, q_vmem, pk_ref, recv_ref, copy_sems, send_sems,
             recv_sem):
        my_x = lax.axis_index("x")
        my_y = lax.axis_index("y")
        my_z = lax.axis_index("z")
        my_lin = my_x * 4 + my_y * 2 + my_z
        gid = my_x * 2 + my_y
        b0 = gid * nb

        kdma = pltpu.make_async_copy(
            k_ref.at[pl.ds(b0, nb)], kv_vmem.at[0], copy_sems.at[0])
        vdma = pltpu.make_async_copy(
            v_ref.at[pl.ds(b0, nb)], kv_vmem.at[1], copy_sems.at[1])
        qdma = pltpu.make_async_copy(
            q_ref.at[pl.ds(b0, nb)], q_vmem, copy_sems.at[2])
        kdma.start()
        vdma.start()
        qdma.start()

        barrier_sem = pltpu.get_barrier_semaphore()
        for peer in range(N_DEV):
            @pl.when(peer != my_lin)
            def _():
                pl.semaphore_signal(
                    barrier_sem, inc=1,
                    device_id=peer, device_id_type=pl.DeviceIdType.LOGICAL,
                )

        qdma.wait()
        kdma.wait()

        q = q_vmem[...]
        k = kv_vmem[0]
        s = jnp.sum(q[:, :, :, None] * k, axis=2) * scale
        m = jnp.max(s, axis=2, keepdims=True)
        p = jnp.exp(s - m)
        l = jnp.sum(p, axis=2, keepdims=True)

        vdma.wait()
        v = kv_vmem[1]
        o = jnp.sum(p[:, :, None, :] * v, axis=3)

        pk_ref[:, :, 0:d] = o
        pk_ref[:, :, d:d + 1] = m
        pk_ref[:, :, d + 1:d + 2] = l

        pl.semaphore_wait(barrier_sem, N_DEV - 1)

        for peer in range(N_DEV):
            @pl.when(peer != my_lin)
            def _():
                rdma = pltpu.make_async_remote_copy(
                    src_ref=pk_ref,
                    dst_ref=recv_ref.at[my_lin],
                    send_sem=send_sems.at[peer],
                    recv_sem=recv_sem,
                    device_id=peer,
                    device_id_type=pl.DeviceIdType.LOGICAL,
                )
                rdma.start()

        for _ in range(N_DEV - 1):
            recv_wait = pltpu.make_async_remote_copy(
                src_ref=pk_ref,
                dst_ref=recv_ref.at[0],
                send_sem=send_sems.at[0],
                recv_sem=recv_sem,
                device_id=0,
                device_id_type=pl.DeviceIdType.LOGICAL,
            )
            recv_wait.wait_recv()

        own = pk_ref[...]
        for g in range(4):
            def slot(s_idx):
                data = recv_ref[s_idx]
                data = jnp.where(s_idx == my_lin, own, data)
                return (data[:, :, 0:d], data[:, :, d:d + 1],
                        data[:, :, d + 1:d + 2])
            o1, m1, l1 = slot(2 * g)
            o2, m2, l2 = slot(2 * g + 1)
            mn = jnp.maximum(m1, m2)
            a1 = jnp.exp(m1 - mn)
            a2 = jnp.exp(m2 - mn)
            ln = a1 * l1 + a2 * l2
            on = (a1 * o1 + a2 * o2) / ln
            out_ref[pl.ds(g * nb, nb)] = on[:, None, :, :]

        for peer in range(N_DEV):
            @pl.when(peer != my_lin)
            def _():
                drain = pltpu.make_async_remote_copy(
                    src_ref=pk_ref,
                    dst_ref=recv_ref.at[my_lin],
                    send_sem=send_sems.at[peer],
                    recv_sem=recv_sem,
                    device_id=peer,
                    device_id_type=pl.DeviceIdType.LOGICAL,
                )
                drain.wait_send()

    return pl.pallas_call(
        body,
        out_shape=jax.ShapeDtypeStruct((b, sq, h, d), jnp.float32),
        in_specs=[
            pl.BlockSpec(memory_space=pltpu.MemorySpace.HBM),
            pl.BlockSpec(memory_space=pltpu.MemorySpace.HBM),
            pl.BlockSpec(memory_space=pltpu.MemorySpace.HBM),
        ],
        out_specs=pl.BlockSpec(memory_space=pltpu.VMEM),
        scratch_shapes=[
            pltpu.VMEM((2, nb, h, d, skv), jnp.float32),
            pltpu.VMEM((nb, h, d), jnp.float32),
            pltpu.VMEM((nb, h, d + 2), jnp.float32),
            pltpu.VMEM((N_DEV, nb, h, d + 2), jnp.float32),
            pltpu.SemaphoreType.DMA((3,)),
            pltpu.SemaphoreType.DMA((N_DEV,)),
            pltpu.SemaphoreType.DMA,
        ],
        compiler_params=pltpu.CompilerParams(collective_id=0),
    )(Qs, Kt, Vt)


# device time: 9610 ns/iter; 1.6706x vs baseline; 1.0010x over previous
import jax
import jax.numpy as jnp
from jax import lax
from jax.experimental import pallas as pl
from jax.experimental.pallas import tpu as pltpu

N_DEV = 8


def kernel(Q, K, V):
    b, sq, h, d = Q.shape
    skv = K.shape[1]
    scale = d ** -0.5
    nb = b // 4

    Kt = jnp.transpose(K, (0, 2, 3, 1))
    Vt = jnp.transpose(V, (0, 2, 3, 1))
    Qs = Q[:, 0, :, :]
    Kt = pltpu.with_memory_space_constraint(Kt, pltpu.MemorySpace.HBM)
    Vt = pltpu.with_memory_space_constraint(Vt, pltpu.MemorySpace.HBM)
    Qs = pltpu.with_memory_space_constraint(Qs, pltpu.MemorySpace.HBM)

    def body(q_ref, k_ref, v_ref, out_ref,
             kv_vmem, q_vmem, pk_ref, recv_ref, copy_sems, send_sems,
             recv_sem):
        my_x = lax.axis_index("x")
        my_y = lax.axis_index("y")
        my_z = lax.axis_index("z")
        my_lin = my_x * 4 + my_y * 2 + my_z
        gid = my_x * 2 + my_y
        b0 = gid * nb

        kdma = pltpu.make_async_copy(
            k_ref.at[pl.ds(b0, nb)], kv_vmem.at[0], copy_sems.at[0])
        vdma = pltpu.make_async_copy(
            v_ref.at[pl.ds(b0, nb)], kv_vmem.at[1], copy_sems.at[1])
        qdma = pltpu.make_async_copy(
            q_ref.at[pl.ds(b0, nb)], q_vmem, copy_sems.at[2])
        kdma.start()
        vdma.start()
        qdma.start()

        barrier_sem = pltpu.get_barrier_semaphore()
        for peer in range(N_DEV):
            @pl.when(peer != my_lin)
            def _():
                pl.semaphore_signal(
                    barrier_sem, inc=1,
                    device_id=peer, device_id_type=pl.DeviceIdType.LOGICAL,
                )
        pl.semaphore_wait(barrier_sem, N_DEV - 1)

        qdma.wait()
        kdma.wait()

        q = q_vmem[...]
        k = kv_vmem[0]
        s = jnp.sum(q[:, :, :, None] * k, axis=2) * scale
        m = jnp.max(s, axis=2, keepdims=True)
        p = jnp.exp(s - m)
        l = jnp.sum(p, axis=2, keepdims=True)

        vdma.wait()
        v = kv_vmem[1]
        o = jnp.sum(p[:, :, None, :] * v, axis=3)

        pk_ref[:, :, 0:d] = o
        pk_ref[:, :, d:d + 1] = m
        pk_ref[:, :, d + 1:d + 2] = l

        for peer in range(N_DEV):
            @pl.when(peer != my_lin)
            def _():
                rdma = pltpu.make_async_remote_copy(
                    src_ref=pk_ref,
                    dst_ref=recv_ref.at[my_lin],
                    send_sem=send_sems.at[peer],
                    recv_sem=recv_sem,
                    device_id=peer,
                    device_id_type=pl.DeviceIdType.LOGICAL,
                )
                rdma.start()

        for _ in range(N_DEV - 1):
            recv_wait = pltpu.make_async_remote_copy(
                src_ref=pk_ref,
                dst_ref=recv_ref.at[0],
                send_sem=send_sems.at[0],
                recv_sem=recv_sem,
                device_id=0,
                device_id_type=pl.DeviceIdType.LOGICAL,
            )
            recv_wait.wait_recv()

        own = pk_ref[...]
        for g in range(4):
            def slot(s_idx):
                data = recv_ref[s_idx]
                data = jnp.where(s_idx == my_lin, own, data)
                return (data[:, :, 0:d], data[:, :, d:d + 1],
                        data[:, :, d + 1:d + 2])
            o1, m1, l1 = slot(2 * g)
            o2, m2, l2 = slot(2 * g + 1)
            mn = jnp.maximum(m1, m2)
            a1 = jnp.exp(m1 - mn)
            a2 = jnp.exp(m2 - mn)
            ln = a1 * l1 + a2 * l2
            on = (a1 * o1 + a2 * o2) / ln
            out_ref[pl.ds(g * nb, nb)] = on[:, None, :, :]

        for peer in range(N_DEV):
            @pl.when(peer != my_lin)
            def _():
                drain = pltpu.make_async_remote_copy(
                    src_ref=pk_ref,
                    dst_ref=recv_ref.at[my_lin],
                    send_sem=send_sems.at[peer],
                    recv_sem=recv_sem,
                    device_id=peer,
                    device_id_type=pl.DeviceIdType.LOGICAL,
                )
                drain.wait_send()

    return pl.pallas_call(
        body,
        out_shape=jax.ShapeDtypeStruct((b, sq, h, d), jnp.float32),
        in_specs=[
            pl.BlockSpec(memory_space=pltpu.MemorySpace.HBM),
            pl.BlockSpec(memory_space=pltpu.MemorySpace.HBM),
            pl.BlockSpec(memory_space=pltpu.MemorySpace.HBM),
        ],
        out_specs=pl.BlockSpec(memory_space=pltpu.VMEM),
        scratch_shapes=[
            pltpu.VMEM((2, nb, h, d, skv), jnp.float32),
            pltpu.VMEM((nb, h, d), jnp.float32),
            pltpu.VMEM((nb, h, d + 2), jnp.float32),
            pltpu.VMEM((N_DEV, nb, h, d + 2), jnp.float32),
            pltpu.SemaphoreType.DMA((3,)),
            pltpu.SemaphoreType.DMA((N_DEV,)),
            pltpu.SemaphoreType.DMA,
        ],
        compiler_params=pltpu.CompilerParams(collective_id=0),
    )(Qs, Kt, Vt)
